# baseline (device time: 190751 ns/iter reference)
import jax
import jax.numpy as jnp
from jax import lax
from jax.experimental import pallas as pl
from jax.experimental.pallas import tpu as pltpu

N_DEV = 32
SQ = 512
D = 1024
DH = 128
H_LOC = 8
SCALE = 0.08838834764831843
CHUNK = SQ // N_DEV


def kernel(x, Wq, Wo, Wk, Wv):
    def body(x_ref, wq_ref, wo_ref, wk_ref, wv_ref, out_ref,
             rs_buf, rs_send, rs_recv, ag_send, ag_recv):
        my = lax.axis_index("i")
        right = jnp.mod(my + 1, N_DEV)

        xm = x_ref[0]
        q = jnp.dot(xm, wq_ref[...], preferred_element_type=jnp.float32)
        k = jnp.dot(xm, wk_ref[...], preferred_element_type=jnp.float32)
        v = jnp.dot(xm, wv_ref[...], preferred_element_type=jnp.float32)
        outs = []
        for h in range(H_LOC):
            qh = q[:, h * DH:(h + 1) * DH]
            kh = k[:, h * DH:(h + 1) * DH]
            vh = v[:, h * DH:(h + 1) * DH]
            s = jnp.dot(qh, kh.T, preferred_element_type=jnp.float32) * SCALE
            m = jnp.max(s, axis=-1, keepdims=True)
            p = jnp.exp(s - m)
            l = jnp.sum(p, axis=-1, keepdims=True)
            outs.append(jnp.dot(p, vh, preferred_element_type=jnp.float32) / l)
        attn = jnp.concatenate(outs, axis=-1)
        out_ref[0] = jnp.dot(attn, wo_ref[...], preferred_element_type=jnp.float32)

        for s_ in range(N_DEV - 1):
            c_send = jnp.mod(my - s_, N_DEV)
            c_recv = jnp.mod(my - s_ - 1, N_DEV)
            rdma = pltpu.make_async_remote_copy(
                src_ref=out_ref.at[0, pl.ds(c_send * CHUNK, CHUNK), :],
                dst_ref=rs_buf.at[s_],
                send_sem=rs_send.at[s_],
                recv_sem=rs_recv.at[s_],
                device_id=(right,),
                device_id_type=pl.DeviceIdType.MESH,
            )
            rdma.start()
            rdma.wait()
            rows = pl.ds(c_recv * CHUNK, CHUNK)
            out_ref[0, rows, :] = out_ref[0, rows, :] + rs_buf[s_]

        for t in range(N_DEV - 1):
            c = jnp.mod(my + 1 - t, N_DEV)
            rows = pl.ds(c * CHUNK, CHUNK)
            rdma = pltpu.make_async_remote_copy(
                src_ref=out_ref.at[0, rows, :],
                dst_ref=out_ref.at[0, rows, :],
                send_sem=ag_send.at[t],
                recv_sem=ag_recv.at[t],
                device_id=(right,),
                device_id_type=pl.DeviceIdType.MESH,
            )
            rdma.start()
            rdma.wait()

    return pl.pallas_call(
        body,
        out_shape=jax.ShapeDtypeStruct((1, SQ, D), jnp.float32),
        in_specs=[pl.BlockSpec(memory_space=pltpu.VMEM)] * 5,
        out_specs=pl.BlockSpec(memory_space=pltpu.VMEM),
        scratch_shapes=[
            pltpu.VMEM((N_DEV - 1, CHUNK, D), jnp.float32),
            pltpu.SemaphoreType.DMA((N_DEV - 1,)),
            pltpu.SemaphoreType.DMA((N_DEV - 1,)),
            pltpu.SemaphoreType.DMA((N_DEV - 1,)),
            pltpu.SemaphoreType.DMA((N_DEV - 1,)),
        ],
    )(x, Wq, Wo, Wk, Wv)


# device time: 84370 ns/iter; 2.2609x vs baseline; 2.2609x over previous
import jax
import jax.numpy as jnp
from jax import lax
from jax.experimental import pallas as pl
from jax.experimental.pallas import tpu as pltpu

N_DEV = 32
SQ = 512
D = 1024
DH = 128
H_LOC = 8
SCALE = 0.08838834764831843
CHUNK = SQ // N_DEV


def kernel(x, Wq, Wo, Wk, Wv):
    def body(x_ref, wq_ref, wo_ref, wk_ref, wv_ref, out_ref,
             rs_buf, rs_send, rs_recv, ag_send, ag_recv):
        my = lax.axis_index("i")

        xm = x_ref[0]
        q = jnp.dot(xm, wq_ref[...], preferred_element_type=jnp.float32)
        k = jnp.dot(xm, wk_ref[...], preferred_element_type=jnp.float32)
        v = jnp.dot(xm, wv_ref[...], preferred_element_type=jnp.float32)
        outs = []
        for h in range(H_LOC):
            qh = q[:, h * DH:(h + 1) * DH]
            kh = k[:, h * DH:(h + 1) * DH]
            vh = v[:, h * DH:(h + 1) * DH]
            s = jnp.dot(qh, kh.T, preferred_element_type=jnp.float32) * SCALE
            m = jnp.max(s, axis=-1, keepdims=True)
            p = jnp.exp(s - m)
            l = jnp.sum(p, axis=-1, keepdims=True)
            outs.append(jnp.dot(p, vh, preferred_element_type=jnp.float32) / l)
        attn = jnp.concatenate(outs, axis=-1)
        out_ref[0] = jnp.dot(attn, wo_ref[...], preferred_element_type=jnp.float32)

        rs_rdmas = []
        for j in range(1, N_DEV):
            d = jnp.mod(my + j, N_DEV)
            rdma = pltpu.make_async_remote_copy(
                src_ref=out_ref.at[0, pl.ds(d * CHUNK, CHUNK), :],
                dst_ref=rs_buf.at[j - 1],
                send_sem=rs_send.at[j - 1],
                recv_sem=rs_recv.at[j - 1],
                device_id=(d,),
                device_id_type=pl.DeviceIdType.MESH,
            )
            rdma.start()
            rs_rdmas.append(rdma)

        my_rows = pl.ds(my * CHUNK, CHUNK)
        for j in range(1, N_DEV):
            rs_rdmas[j - 1].wait_recv()
            out_ref[0, my_rows, :] = out_ref[0, my_rows, :] + rs_buf[j - 1]

        ag_rdmas = []
        for j in range(1, N_DEV):
            d = jnp.mod(my + j, N_DEV)
            rdma = pltpu.make_async_remote_copy(
                src_ref=out_ref.at[0, my_rows, :],
                dst_ref=out_ref.at[0, my_rows, :],
                send_sem=ag_send.at[j - 1],
                recv_sem=ag_recv.at[j - 1],
                device_id=(d,),
                device_id_type=pl.DeviceIdType.MESH,
            )
            rdma.start()
            ag_rdmas.append(rdma)

        for j in range(1, N_DEV):
            ag_rdmas[j - 1].wait_recv()
        for j in range(1, N_DEV):
            rs_rdmas[j - 1].wait_send()
            ag_rdmas[j - 1].wait_send()

    return pl.pallas_call(
        body,
        out_shape=jax.ShapeDtypeStruct((1, SQ, D), jnp.float32),
        in_specs=[pl.BlockSpec(memory_space=pltpu.VMEM)] * 5,
        out_specs=pl.BlockSpec(memory_space=pltpu.VMEM),
        scratch_shapes=[
            pltpu.VMEM((N_DEV - 1, CHUNK, D), jnp.float32),
            pltpu.SemaphoreType.DMA((N_DEV - 1,)),
            pltpu.SemaphoreType.DMA((N_DEV - 1,)),
            pltpu.SemaphoreType.DMA((N_DEV - 1,)),
            pltpu.SemaphoreType.DMA((N_DEV - 1,)),
        ],
    )(x, Wq, Wo, Wk, Wv)


# device time: 60342 ns/iter; 3.1612x vs baseline; 1.3982x over previous
import jax
import jax.numpy as jnp
from jax import lax
from jax.experimental import pallas as pl
from jax.experimental.pallas import tpu as pltpu

N_DEV = 32
SQ = 512
D = 1024
DH = 128
H_LOC = 8
SCALE = 0.08838834764831843
CHUNK = SQ // N_DEV

BF = jnp.bfloat16
F32 = jnp.float32


def _dot(a, b):
    return jnp.dot(a.astype(BF), b.astype(BF), preferred_element_type=F32)


def kernel(x, Wq, Wo, Wk, Wv):
    def body(x_ref, wq_ref, wo_ref, wk_ref, wv_ref, out_ref,
             part16, rs_buf, ag_buf, bcast_src,
             rs_send, rs_recv, ag_send, ag_recv):
        my = lax.axis_index("i")

        xm = x_ref[0]
        q = _dot(xm, wq_ref[...])
        k = _dot(xm, wk_ref[...])
        v = _dot(xm, wv_ref[...])
        outs = []
        for h in range(H_LOC):
            qh = q[:, h * DH:(h + 1) * DH]
            kh = k[:, h * DH:(h + 1) * DH]
            vh = v[:, h * DH:(h + 1) * DH]
            s = _dot(qh, kh.T) * SCALE
            m = jnp.max(s, axis=-1, keepdims=True)
            p = jnp.exp(s - m)
            l = jnp.sum(p, axis=-1, keepdims=True)
            outs.append(_dot(p, vh) / l)
        attn = jnp.concatenate(outs, axis=-1)
        partial = _dot(attn, wo_ref[...])
        out_ref[0] = partial
        part16[...] = partial.astype(BF)

        rs_rdmas = []
        for j in range(1, N_DEV):
            d = jnp.mod(my + j, N_DEV)
            rdma = pltpu.make_async_remote_copy(
                src_ref=part16.at[pl.ds(d * CHUNK, CHUNK), :],
                dst_ref=rs_buf.at[j - 1],
                send_sem=rs_send.at[j - 1],
                recv_sem=rs_recv.at[j - 1],
                device_id=(d,),
                device_id_type=pl.DeviceIdType.MESH,
            )
            rdma.start()
            rs_rdmas.append(rdma)

        my_rows = pl.ds(my * CHUNK, CHUNK)
        for j in range(1, N_DEV):
            rs_rdmas[j - 1].wait_recv()
            out_ref[0, my_rows, :] = (
                out_ref[0, my_rows, :] + rs_buf[j - 1].astype(F32)
            )
        bcast_src[...] = out_ref[0, my_rows, :].astype(BF)

        ag_rdmas = []
        for j in range(1, N_DEV):
            d = jnp.mod(my + j, N_DEV)
            rdma = pltpu.make_async_remote_copy(
                src_ref=bcast_src,
                dst_ref=ag_buf.at[j - 1],
                send_sem=ag_send.at[j - 1],
                recv_sem=ag_recv.at[j - 1],
                device_id=(d,),
                device_id_type=pl.DeviceIdType.MESH,
            )
            rdma.start()
            ag_rdmas.append(rdma)

        for j in range(1, N_DEV):
            ag_rdmas[j - 1].wait_recv()
            src = jnp.mod(my - j, N_DEV)
            out_ref[0, pl.ds(src * CHUNK, CHUNK), :] = ag_buf[j - 1].astype(F32)

        for j in range(1, N_DEV):
            rs_rdmas[j - 1].wait_send()
            ag_rdmas[j - 1].wait_send()

    return pl.pallas_call(
        body,
        out_shape=jax.ShapeDtypeStruct((1, SQ, D), F32),
        in_specs=[pl.BlockSpec(memory_space=pltpu.VMEM)] * 5,
        out_specs=pl.BlockSpec(memory_space=pltpu.VMEM),
        scratch_shapes=[
            pltpu.VMEM((SQ, D), BF),
            pltpu.VMEM((N_DEV - 1, CHUNK, D), BF),
            pltpu.VMEM((N_DEV - 1, CHUNK, D), BF),
            pltpu.VMEM((CHUNK, D), BF),
            pltpu.SemaphoreType.DMA((N_DEV - 1,)),
            pltpu.SemaphoreType.DMA((N_DEV - 1,)),
            pltpu.SemaphoreType.DMA((N_DEV - 1,)),
            pltpu.SemaphoreType.DMA((N_DEV - 1,)),
        ],
    )(x, Wq, Wo, Wk, Wv)


# device time: 59691 ns/iter; 3.1956x vs baseline; 1.0109x over previous
import jax
import jax.numpy as jnp
from jax import lax
from jax.experimental import pallas as pl
from jax.experimental.pallas import tpu as pltpu

N_DEV = 32
SQ = 512
D = 1024
DH = 128
H_LOC = 8
SCALE = 0.08838834764831843
CHUNK = SQ // N_DEV

BF = jnp.bfloat16
F32 = jnp.float32


def kernel(x, Wq, Wo, Wk, Wv):
    def body(x_ref, wq_ref, wo_ref, wk_ref, wv_ref, out_ref,
             part16, rs_buf, ag_buf, bcast_src,
             rs_send, rs_recv, ag_send, ag_recv):
        my = lax.axis_index("i")

        with jax.named_scope("compute"):
            x16 = x_ref[0].astype(BF)
            q = jnp.dot(
                x16, wq_ref[...].astype(BF), preferred_element_type=F32
            ).astype(BF)
            k = jnp.dot(
                x16, wk_ref[...].astype(BF), preferred_element_type=F32
            ).astype(BF)
            v = jnp.dot(
                x16, wv_ref[...].astype(BF), preferred_element_type=F32
            ).astype(BF)
            outs = []
            for h in range(H_LOC):
                qh = q[:, h * DH:(h + 1) * DH]
                kh = k[:, h * DH:(h + 1) * DH]
                vh = v[:, h * DH:(h + 1) * DH]
                s = lax.dot_general(
                    qh, kh, (((1,), (1,)), ((), ())),
                    preferred_element_type=F32,
                ) * SCALE
                m = jnp.max(s, axis=-1, keepdims=True)
                p = jnp.exp(s - m)
                l = jnp.sum(p, axis=-1, keepdims=True)
                o = lax.dot_general(
                    p.astype(BF), vh, (((1,), (0,)), ((), ())),
                    preferred_element_type=F32,
                ) / l
                outs.append(o.astype(BF))
            attn = jnp.concatenate(outs, axis=-1)
            partial = jnp.dot(
                attn, wo_ref[...].astype(BF), preferred_element_type=F32
            )
            part16[...] = partial.astype(BF)

        with jax.named_scope("pA_send"):
            rs_rdmas = []
            for j in range(1, N_DEV):
                d = jnp.mod(my + j, N_DEV)
                rdma = pltpu.make_async_remote_copy(
                    src_ref=part16.at[pl.ds(d * CHUNK, CHUNK), :],
                    dst_ref=rs_buf.at[my],
                    send_sem=rs_send.at[j - 1],
                    recv_sem=rs_recv.at[my],
                    device_id=(d,),
                    device_id_type=pl.DeviceIdType.MESH,
                )
                rdma.start()
                rs_rdmas.append(rdma)
            rs_buf[my] = part16[pl.ds(my * CHUNK, CHUNK), :]

        with jax.named_scope("pA_wait"):
            for j in range(1, N_DEV):
                s_id = jnp.mod(my + j, N_DEV)
                pltpu.make_async_remote_copy(
                    src_ref=part16.at[pl.ds(0, CHUNK), :],
                    dst_ref=rs_buf.at[s_id],
                    send_sem=rs_send.at[j - 1],
                    recv_sem=rs_recv.at[s_id],
                    device_id=(s_id,),
                    device_id_type=pl.DeviceIdType.MESH,
                ).wait_recv()

        with jax.named_scope("reduce"):
            reduced = jnp.sum(rs_buf[...].astype(F32), axis=0)
            bcast_src[...] = reduced.astype(BF)
            ag_buf[my] = bcast_src[...]

        with jax.named_scope("pB_send"):
            ag_rdmas = []
            for j in range(1, N_DEV):
                d = jnp.mod(my + j, N_DEV)
                rdma = pltpu.make_async_remote_copy(
                    src_ref=bcast_src,
                    dst_ref=ag_buf.at[my],
                    send_sem=ag_send.at[j - 1],
                    recv_sem=ag_recv.at[my],
                    device_id=(d,),
                    device_id_type=pl.DeviceIdType.MESH,
                )
                rdma.start()
                ag_rdmas.append(rdma)

        with jax.named_scope("pB_wait"):
            for j in range(1, N_DEV):
                s_id = jnp.mod(my + j, N_DEV)
                pltpu.make_async_remote_copy(
                    src_ref=bcast_src,
                    dst_ref=ag_buf.at[s_id],
                    send_sem=ag_send.at[j - 1],
                    recv_sem=ag_recv.at[s_id],
                    device_id=(s_id,),
                    device_id_type=pl.DeviceIdType.MESH,
                ).wait_recv()

        with jax.named_scope("store"):
            out_ref[0] = ag_buf[...].reshape(SQ, D).astype(F32)

        with jax.named_scope("drain"):
            for j in range(1, N_DEV):
                rs_rdmas[j - 1].wait_send()
                ag_rdmas[j - 1].wait_send()

    return pl.pallas_call(
        body,
        out_shape=jax.ShapeDtypeStruct((1, SQ, D), F32),
        in_specs=[pl.BlockSpec(memory_space=pltpu.VMEM)] * 5,
        out_specs=pl.BlockSpec(memory_space=pltpu.VMEM),
        scratch_shapes=[
            pltpu.VMEM((SQ, D), BF),
            pltpu.VMEM((N_DEV, CHUNK, D), BF),
            pltpu.VMEM((N_DEV, CHUNK, D), BF),
            pltpu.VMEM((CHUNK, D), BF),
            pltpu.SemaphoreType.DMA((N_DEV - 1,)),
            pltpu.SemaphoreType.DMA((N_DEV,)),
            pltpu.SemaphoreType.DMA((N_DEV - 1,)),
            pltpu.SemaphoreType.DMA((N_DEV,)),
        ],
    )(x, Wq, Wo, Wk, Wv)


# device time: 59685 ns/iter; 3.1960x vs baseline; 1.0001x over previous
import jax
import jax.numpy as jnp
from jax import lax
from jax.experimental import pallas as pl
from jax.experimental.pallas import tpu as pltpu

N_DEV = 32
SQ = 512
D = 1024
DH = 128
H_LOC = 8
SCALE = 0.08838834764831843
CHUNK = SQ // N_DEV

BF = jnp.bfloat16
F32 = jnp.float32


def kernel(x, Wq, Wo, Wk, Wv):
    def body(x_ref, wq_ref, wo_ref, wk_ref, wv_ref, out_ref,
             part16, rs_buf, ag_buf, bcast_src,
             rs_send, rs_recv, ag_send, ag_recv):
        my = lax.axis_index("i")

        x16 = x_ref[0].astype(BF)
        q = jnp.dot(
            x16, wq_ref[...].astype(BF), preferred_element_type=F32
        ).astype(BF)
        k = jnp.dot(
            x16, wk_ref[...].astype(BF), preferred_element_type=F32
        ).astype(BF)
        v = jnp.dot(
            x16, wv_ref[...].astype(BF), preferred_element_type=F32
        ).astype(BF)
        outs = []
        for h in range(H_LOC):
            qh = q[:, h * DH:(h + 1) * DH]
            kh = k[:, h * DH:(h + 1) * DH]
            vh = v[:, h * DH:(h + 1) * DH]
            s = lax.dot_general(
                qh, kh, (((1,), (1,)), ((), ())), preferred_element_type=F32
            ) * SCALE
            m = jnp.max(s, axis=-1, keepdims=True)
            p = jnp.exp(s - m)
            l = jnp.sum(p, axis=-1, keepdims=True)
            o = lax.dot_general(
                p.astype(BF), vh, (((1,), (0,)), ((), ())),
                preferred_element_type=F32,
            ) / l
            outs.append(o.astype(BF))
        attn = jnp.concatenate(outs, axis=-1)
        partial = jnp.dot(
            attn, wo_ref[...].astype(BF), preferred_element_type=F32
        )
        part16[...] = partial.astype(BF)

        rs_rdmas = []
        for j in range(1, N_DEV):
            d = jnp.mod(my + j, N_DEV)
            rdma = pltpu.make_async_remote_copy(
                src_ref=part16.at[pl.ds(d * CHUNK, CHUNK), :],
                dst_ref=rs_buf.at[my],
                send_sem=rs_send.at[j - 1],
                recv_sem=rs_recv.at[my],
                device_id=(d,),
                device_id_type=pl.DeviceIdType.MESH,
            )
            rdma.start()
            rs_rdmas.append(rdma)
        rs_buf[my] = part16[pl.ds(my * CHUNK, CHUNK), :]

        for j in range(1, N_DEV):
            s_id = jnp.mod(my + j, N_DEV)
            pltpu.make_async_remote_copy(
                src_ref=part16.at[pl.ds(0, CHUNK), :],
                dst_ref=rs_buf.at[s_id],
                send_sem=rs_send.at[j - 1],
                recv_sem=rs_recv.at[s_id],
                device_id=(s_id,),
                device_id_type=pl.DeviceIdType.MESH,
            ).wait_recv()
        reduced = jnp.sum(rs_buf[...].astype(F32), axis=0)
        bcast_src[...] = reduced.astype(BF)
        ag_buf[my] = bcast_src[...]

        ag_rdmas = []
        for j in range(1, N_DEV):
            d = jnp.mod(my + j, N_DEV)
            rdma = pltpu.make_async_remote_copy(
                src_ref=bcast_src,
                dst_ref=ag_buf.at[my],
                send_sem=ag_send.at[j - 1],
                recv_sem=ag_recv.at[my],
                device_id=(d,),
                device_id_type=pl.DeviceIdType.MESH,
            )
            rdma.start()
            ag_rdmas.append(rdma)

        for j in range(1, N_DEV):
            s_id = jnp.mod(my + j, N_DEV)
            pltpu.make_async_remote_copy(
                src_ref=bcast_src,
                dst_ref=ag_buf.at[s_id],
                send_sem=ag_send.at[j - 1],
                recv_sem=ag_recv.at[s_id],
                device_id=(s_id,),
                device_id_type=pl.DeviceIdType.MESH,
            ).wait_recv()
        out_ref[0] = ag_buf[...].reshape(SQ, D).astype(F32)

        for j in range(1, N_DEV):
            rs_rdmas[j - 1].wait_send()
            ag_rdmas[j - 1].wait_send()

    return pl.pallas_call(
        body,
        out_shape=jax.ShapeDtypeStruct((1, SQ, D), F32),
        in_specs=[pl.BlockSpec(memory_space=pltpu.VMEM)] * 5,
        out_specs=pl.BlockSpec(memory_space=pltpu.VMEM),
        scratch_shapes=[
            pltpu.VMEM((SQ, D), BF),
            pltpu.VMEM((N_DEV, CHUNK, D), BF),
            pltpu.VMEM((N_DEV, CHUNK, D), BF),
            pltpu.VMEM((CHUNK, D), BF),
            pltpu.SemaphoreType.DMA((N_DEV - 1,)),
            pltpu.SemaphoreType.DMA((N_DEV,)),
            pltpu.SemaphoreType.DMA((N_DEV - 1,)),
            pltpu.SemaphoreType.DMA((N_DEV,)),
        ],
    )(x, Wq, Wo, Wk, Wv)
